# Initial kernel scaffold; baseline (speedup 1.0000x reference)
import jax
import jax.numpy as jnp
from jax.experimental import pallas as pl


def kernel(x_in):
    raise NotImplementedError("write your pallas kernel here")


# speedup vs baseline: 10.6996x; 10.6996x over previous
import functools

import jax
import jax.numpy as jnp
import numpy as np
from jax.experimental import pallas as pl

K = 16
T = 8
TB = 32


def _mix_matrix(F: int) -> np.ndarray:
    series = np.arange(1, K + 1)
    omega = np.arange(F)
    idx = np.clip(omega[:, None] * series[None, :], 0, F - 1)
    M = np.zeros((F, F), dtype=np.float32)
    np.add.at(M, (np.repeat(omega, K), idx.reshape(-1)), 1.0)
    return M


def _hcf_kernel(xprev_ref, xcur_ref, xnext_ref, mblk_ref, out_ref, *, nt):
    tc = pl.program_id(1)
    xe = jnp.concatenate(
        [xprev_ref[0], xcur_ref[0], xnext_ref[0]], axis=0
    )
    tglob = (tc * TB - TB) + jax.lax.broadcasted_iota(jnp.int32, (3 * TB, 1, 1), 0)
    valid = (tglob >= 0) & (tglob < nt * TB)
    xe = jnp.where(valid, xe, 0.0)
    s2 = xe[:-1] + xe[1:]
    s4 = s2[:-2] + s2[2:]
    s8 = s4[:-4] + s4[4:]
    s16 = s8[:-8] + s8[8:]
    win = s16[TB - T : 2 * TB - T] + xe[TB + T : 2 * TB + T]
    F = win.shape[1]
    C = win.shape[2]
    w2 = win.reshape(TB // 2, 2 * F, C)
    mblk = mblk_ref[...]
    outs = [
        jax.lax.dot(mblk, w2[i], preferred_element_type=jnp.float32)
        for i in range(TB // 2)
    ]
    out_ref[...] = jnp.stack(outs).reshape(1, TB, F, C)


def kernel(x_in):
    B, Tt, F, C = x_in.shape
    nt = Tt // TB
    M = _mix_matrix(F)
    mblk = np.zeros((2 * F, 2 * F), dtype=np.float32)
    mblk[:F, :F] = M
    mblk[F:, F:] = M
    mblk = jnp.asarray(mblk)

    grid = (B, nt)
    xspec = lambda fn: pl.BlockSpec((1, TB, F, C), fn)
    return pl.pallas_call(
        functools.partial(_hcf_kernel, nt=nt),
        grid=grid,
        in_specs=[
            xspec(lambda b, t: (b, jnp.maximum(t - 1, 0), 0, 0)),
            xspec(lambda b, t: (b, t, 0, 0)),
            xspec(lambda b, t: (b, jnp.minimum(t + 1, nt - 1), 0, 0)),
            pl.BlockSpec((2 * F, 2 * F), lambda b, t: (0, 0)),
        ],
        out_specs=pl.BlockSpec((1, TB, F, C), lambda b, t: (b, t, 0, 0)),
        out_shape=jax.ShapeDtypeStruct((B, Tt, F, C), x_in.dtype),
    )(x_in, x_in, x_in, mblk)
